# baseline (speedup 1.0000x reference)
import functools

import jax
import jax.numpy as jnp
from jax import lax
from jax.experimental import pallas as pl
from jax.experimental.pallas import tpu as pltpu
from jax.experimental.pallas import tpu_sc as plsc

IMAGE_SIZE = 300
D_POS = 64
DG = 128
T = 100
B = 1024
ROWS = B * T
BBOX_DIM = 4 * D_POS
OUT_DIM = 512
ZVOFF = 304

CHUNK_R = 128
GBLK = 128
NW = 32
R_PER_W = ROWS // NW
NCHUNK = R_PER_W // CHUNK_R
UNITS = NW * NCHUNK
MB = 1600


def _sc_gather_body(pk_hbm, table_hbm, poe_hbm, pk_v, idx_v, rows_v, sem):
    wid = lax.axis_index("s") * 2 + lax.axis_index("c")
    r0 = wid * R_PER_W

    def chunk(ci, carry):
        off = r0 + ci * CHUNK_R
        pltpu.sync_copy(pk_hbm.at[:, pl.ds(off, CHUNK_R)], pk_v)
        for c in range(4):
            for v in range(CHUNK_R // 16):
                f = lax.bitcast_convert_type(
                    pk_v[c, pl.ds(v * 16, 16)], jnp.float32
                )
                o = pk_v[4, pl.ds(v * 16, 16)]
                xi = (f * float(IMAGE_SIZE)).astype(jnp.int32)
                xi = jnp.minimum(jnp.maximum(xi, 0), IMAGE_SIZE)
                xi = jnp.where(o == 1, xi, xi + ZVOFF)
                idx_v[c, pl.ds(v * 16, 16)] = xi
        descs = [
            pltpu.async_copy(table_hbm.at[idx_v.at[c]], rows_v.at[c], sem)
            for c in range(4)
        ]
        for d in descs:
            d.wait()
        for c in range(4):
            pltpu.sync_copy(
                rows_v.at[c],
                poe_hbm.at[wid, c // 2, pl.ds(ci * CHUNK_R, CHUNK_R),
                           pl.ds(D_POS * (c % 2), D_POS)],
            )
        return carry

    lax.fori_loop(0, NCHUNK, chunk, 0)


def _sc_gather(packed, table):
    mesh = plsc.VectorSubcoreMesh(core_axis_name="c", subcore_axis_name="s")
    return functools.partial(
        pl.kernel,
        mesh=mesh,
        compiler_params=pltpu.CompilerParams(use_tc_tiling_on_sc=False),
        out_type=jax.ShapeDtypeStruct(
            (NW, 2, R_PER_W, DG), jnp.float32),
        scratch_types=[
            pltpu.VMEM((5, CHUNK_R), jnp.int32),
            pltpu.VMEM((4, GBLK), jnp.int32),
            pltpu.VMEM((4, GBLK, D_POS), jnp.float32),
            pltpu.SemaphoreType.DMA,
        ],
    )(_sc_gather_body)(packed, table)


def _base_body(wt_ref, ww_ref, b_ref, out_ref):
    acc = (
        jnp.dot(wt_ref[...], ww_ref[...], preferred_element_type=jnp.float32)
        + b_ref[...]
    )
    for k in range(MB // T):
        out_ref[pl.ds(k * T, T), :] = acc


def _mm_body(poe_ref, wq_ref, base_ref, out_ref):
    acc = base_ref[...]
    for p in range(2):
        acc = acc + jnp.dot(
            poe_ref[0, p], wq_ref[p], preferred_element_type=jnp.float32
        )
    out_ref[...] = jnp.maximum(acc, 0.0)


def kernel(batch_fractional_bboxs, batch_obj_vecs, pos_table, word_table, W, b):
    frac_t = batch_fractional_bboxs.reshape(ROWS, 4).T
    packed = jnp.concatenate(
        [lax.bitcast_convert_type(frac_t, jnp.int32),
         batch_obj_vecs.reshape(1, ROWS)], axis=0)
    table = jnp.pad(pos_table, ((0, 3 + ZVOFF), (0, 0)))
    Wq = W[:BBOX_DIM].reshape(2, DG, OUT_DIM)
    Ww = W[BBOX_DIM:]

    base_rep = pl.pallas_call(
        _base_body,
        out_shape=jax.ShapeDtypeStruct((MB, OUT_DIM), jnp.float32),
    )(word_table, Ww, b.reshape(1, OUT_DIM))

    poe_r = _sc_gather(packed, table)

    nh = R_PER_W // MB
    return pl.pallas_call(
        _mm_body,
        grid=(NW, nh),
        in_specs=[
            pl.BlockSpec((1, 2, MB, DG), lambda w, h: (w, 0, h, 0)),
            pl.BlockSpec((2, DG, OUT_DIM), lambda w, h: (0, 0, 0)),
            pl.BlockSpec((MB, OUT_DIM), lambda w, h: (0, 0)),
        ],
        out_specs=pl.BlockSpec((MB, OUT_DIM),
                               lambda w, h: (w * nh + h, 0)),
        out_shape=jax.ShapeDtypeStruct((ROWS, OUT_DIM), jnp.float32),
    )(poe_r, Wq, base_rep)


# speedup vs baseline: 1.0411x; 1.0411x over previous
import functools

import jax
import jax.numpy as jnp
from jax import lax
from jax.experimental import pallas as pl
from jax.experimental.pallas import tpu as pltpu
from jax.experimental.pallas import tpu_sc as plsc

IMAGE_SIZE = 300
D_POS = 64
DG = 128
T = 100
B = 1024
ROWS = B * T
BBOX_DIM = 4 * D_POS
OUT_DIM = 512
ZVOFF = 304

NW = 32
SLICES = 2
ROWS_S = ROWS // SLICES
RT = ROWS_S // NW
CHUNK_R = 80
NCHUNK = RT // CHUNK_R


def _sc_gather_body(pk_hbm, table_hbm, poe_hbm, pk_v, idx_v, rows_v, sem):
    wid = lax.axis_index("s") * 2 + lax.axis_index("c")
    r0 = wid * RT

    def chunk(ci, carry):
        off = r0 + ci * CHUNK_R
        pltpu.sync_copy(pk_hbm.at[:, pl.ds(off, CHUNK_R)], pk_v)
        for c in range(4):
            for v in range(CHUNK_R // 16):
                f = lax.bitcast_convert_type(
                    pk_v[c, pl.ds(v * 16, 16)], jnp.float32
                )
                o = pk_v[4, pl.ds(v * 16, 16)]
                xi = (f * float(IMAGE_SIZE)).astype(jnp.int32)
                xi = jnp.minimum(jnp.maximum(xi, 0), IMAGE_SIZE)
                xi = jnp.where(o == 1, xi, xi + ZVOFF)
                idx_v[c, pl.ds(v * 16, 16)] = xi
        descs = [
            pltpu.async_copy(table_hbm.at[idx_v.at[c]], rows_v.at[c], sem)
            for c in range(4)
        ]
        for d in descs:
            d.wait()
        for c in range(4):
            pltpu.sync_copy(
                rows_v.at[c],
                poe_hbm.at[wid, c // 2, pl.ds(ci * CHUNK_R, CHUNK_R),
                           pl.ds(D_POS * (c % 2), D_POS)],
            )
        return carry

    lax.fori_loop(0, NCHUNK, chunk, 0)


def _sc_gather(packed_s, table):
    mesh = plsc.VectorSubcoreMesh(core_axis_name="c", subcore_axis_name="s")
    return functools.partial(
        pl.kernel,
        mesh=mesh,
        compiler_params=pltpu.CompilerParams(use_tc_tiling_on_sc=False),
        out_type=jax.ShapeDtypeStruct((NW, 2, RT, DG), jnp.float32),
        scratch_types=[
            pltpu.VMEM((5, CHUNK_R), jnp.int32),
            pltpu.VMEM((4, CHUNK_R), jnp.int32),
            pltpu.VMEM((4, CHUNK_R, D_POS), jnp.float32),
            pltpu.SemaphoreType.DMA,
        ],
    )(_sc_gather_body)(packed_s, table)


def _base_body(wt_ref, ww_ref, b_ref, out_ref):
    acc = (
        jnp.dot(wt_ref[...], ww_ref[...], preferred_element_type=jnp.float32)
        + b_ref[...]
    )
    for k in range(RT // T):
        out_ref[pl.ds(k * T, T), :] = acc


def _mm_body(poe_ref, wq_ref, base_ref, out_ref):
    acc = base_ref[...]
    for p in range(2):
        acc = acc + jnp.dot(
            poe_ref[0, p], wq_ref[p], preferred_element_type=jnp.float32
        )
    out_ref[...] = jnp.maximum(acc, 0.0)


def _mm_body_alias(prev_ref, poe_ref, wq_ref, base_ref, out_ref):
    _mm_body(poe_ref, wq_ref, base_ref, out_ref)


def kernel(batch_fractional_bboxs, batch_obj_vecs, pos_table, word_table, W, b):
    frac_t = batch_fractional_bboxs.reshape(ROWS, 4).T
    packed = jnp.concatenate(
        [lax.bitcast_convert_type(frac_t, jnp.int32),
         batch_obj_vecs.reshape(1, ROWS)], axis=0)
    table = jnp.pad(pos_table, ((0, 3 + ZVOFF), (0, 0)))
    Wq = W[:BBOX_DIM].reshape(2, DG, OUT_DIM)
    Ww = W[BBOX_DIM:]

    base_rep = pl.pallas_call(
        _base_body,
        out_shape=jax.ShapeDtypeStruct((RT, OUT_DIM), jnp.float32),
    )(word_table, Ww, b.reshape(1, OUT_DIM))

    poe_s = [
        _sc_gather(packed[:, s * ROWS_S:(s + 1) * ROWS_S], table)
        for s in range(SLICES)
    ]

    mm_specs = [
        pl.BlockSpec((1, 2, RT, DG), lambda t: (t, 0, 0, 0)),
        pl.BlockSpec((2, DG, OUT_DIM), lambda t: (0, 0, 0)),
        pl.BlockSpec((RT, OUT_DIM), lambda t: (0, 0)),
    ]
    out = pl.pallas_call(
        _mm_body,
        grid=(NW,),
        in_specs=mm_specs,
        out_specs=pl.BlockSpec((RT, OUT_DIM), lambda t: (t, 0)),
        out_shape=jax.ShapeDtypeStruct((ROWS, OUT_DIM), jnp.float32),
    )(poe_s[0], Wq, base_rep)
    for s in range(1, SLICES):
        out = pl.pallas_call(
            _mm_body_alias,
            grid=(NW,),
            in_specs=[pl.BlockSpec(memory_space=pltpu.MemorySpace.HBM)]
            + mm_specs,
            out_specs=pl.BlockSpec((RT, OUT_DIM),
                                   lambda t, s=s: (s * NW + t, 0)),
            out_shape=jax.ShapeDtypeStruct((ROWS, OUT_DIM), jnp.float32),
            input_output_aliases={0: 0},
        )(out, poe_s[s], Wq, base_rep)
    return out


# speedup vs baseline: 1.0448x; 1.0036x over previous
import functools

import jax
import jax.numpy as jnp
from jax import lax
from jax.experimental import pallas as pl
from jax.experimental.pallas import tpu as pltpu
from jax.experimental.pallas import tpu_sc as plsc

IMAGE_SIZE = 300
D_POS = 64
DG = 128
T = 100
B = 1024
ROWS = B * T
BBOX_DIM = 4 * D_POS
OUT_DIM = 512
ZVOFF = 304

NW = 32
SLICES = 2
ROWS_S = ROWS // SLICES
RT = ROWS_S // NW
CHUNK_R = 80
NCHUNK = RT // CHUNK_R


def _sc_gather_body(pk_hbm, table_hbm, poe_hbm, pk_v, idx_v, rows_v, sem):
    wid = lax.axis_index("s") * 2 + lax.axis_index("c")
    r0 = wid * RT

    def load_fire(ci, bb):
        off = r0 + ci * CHUNK_R
        pltpu.sync_copy(pk_hbm.at[:, pl.ds(off, CHUNK_R)], pk_v.at[bb])
        for c in range(4):
            for v in range(CHUNK_R // 16):
                f = lax.bitcast_convert_type(
                    pk_v[bb, c, pl.ds(v * 16, 16)], jnp.float32
                )
                o = pk_v[bb, 4, pl.ds(v * 16, 16)]
                xi = (f * float(IMAGE_SIZE)).astype(jnp.int32)
                xi = jnp.minimum(jnp.maximum(xi, 0), IMAGE_SIZE)
                xi = jnp.where(o == 1, xi, xi + ZVOFF)
                idx_v[bb, c, pl.ds(v * 16, 16)] = xi
        for c in range(4):
            pltpu.async_copy(
                table_hbm.at[idx_v.at[bb, c]], rows_v.at[bb, c], sem)

    def drain_scatter(ci, bb):
        for c in range(4):
            pltpu.make_async_copy(
                table_hbm.at[idx_v.at[bb, c]], rows_v.at[bb, c], sem).wait()
        for c in range(4):
            pltpu.sync_copy(
                rows_v.at[bb, c],
                poe_hbm.at[wid, c // 2, pl.ds(ci * CHUNK_R, CHUNK_R),
                           pl.ds(D_POS * (c % 2), D_POS)],
            )

    load_fire(0, 0)

    def pair(j, carry):
        c0 = 2 * j
        load_fire(c0 + 1, 1)
        drain_scatter(c0, 0)
        load_fire(c0 + 2, 0)
        drain_scatter(c0 + 1, 1)
        return carry

    lax.fori_loop(0, NCHUNK // 2 - 1, pair, 0)
    load_fire(NCHUNK - 1, 1)
    drain_scatter(NCHUNK - 2, 0)
    drain_scatter(NCHUNK - 1, 1)


def _sc_gather(packed_s, table):
    mesh = plsc.VectorSubcoreMesh(core_axis_name="c", subcore_axis_name="s")
    return functools.partial(
        pl.kernel,
        mesh=mesh,
        compiler_params=pltpu.CompilerParams(use_tc_tiling_on_sc=False),
        out_type=jax.ShapeDtypeStruct((NW, 2, RT, DG), jnp.float32),
        scratch_types=[
            pltpu.VMEM((2, 5, CHUNK_R), jnp.int32),
            pltpu.VMEM((2, 4, CHUNK_R), jnp.int32),
            pltpu.VMEM((2, 4, CHUNK_R, D_POS), jnp.float32),
            pltpu.SemaphoreType.DMA,
        ],
    )(_sc_gather_body)(packed_s, table)


def _base_body(wt_ref, ww_ref, b_ref, out_ref):
    acc = (
        jnp.dot(wt_ref[...], ww_ref[...], preferred_element_type=jnp.float32)
        + b_ref[...]
    )
    for k in range(RT // T):
        out_ref[pl.ds(k * T, T), :] = acc


def _mm_body(poe_ref, wq_ref, base_ref, out_ref):
    acc = base_ref[...]
    for p in range(2):
        acc = acc + jnp.dot(
            poe_ref[0, p], wq_ref[p], preferred_element_type=jnp.float32
        )
    out_ref[...] = jnp.maximum(acc, 0.0)


def _mm_body_alias(prev_ref, poe_ref, wq_ref, base_ref, out_ref):
    _mm_body(poe_ref, wq_ref, base_ref, out_ref)


def kernel(batch_fractional_bboxs, batch_obj_vecs, pos_table, word_table, W, b):
    frac_t = batch_fractional_bboxs.reshape(ROWS, 4).T
    packed = jnp.concatenate(
        [lax.bitcast_convert_type(frac_t, jnp.int32),
         batch_obj_vecs.reshape(1, ROWS)], axis=0)
    table = jnp.pad(pos_table, ((0, 3 + ZVOFF), (0, 0)))
    Wq = W[:BBOX_DIM].reshape(2, DG, OUT_DIM)
    Ww = W[BBOX_DIM:]

    base_rep = pl.pallas_call(
        _base_body,
        out_shape=jax.ShapeDtypeStruct((RT, OUT_DIM), jnp.float32),
    )(word_table, Ww, b.reshape(1, OUT_DIM))

    poe_s = [
        _sc_gather(packed[:, s * ROWS_S:(s + 1) * ROWS_S], table)
        for s in range(SLICES)
    ]

    mm_specs = [
        pl.BlockSpec((1, 2, RT, DG), lambda t: (t, 0, 0, 0)),
        pl.BlockSpec((2, DG, OUT_DIM), lambda t: (0, 0, 0)),
        pl.BlockSpec((RT, OUT_DIM), lambda t: (0, 0)),
    ]
    out = pl.pallas_call(
        _mm_body,
        grid=(NW,),
        in_specs=mm_specs,
        out_specs=pl.BlockSpec((RT, OUT_DIM), lambda t: (t, 0)),
        out_shape=jax.ShapeDtypeStruct((ROWS, OUT_DIM), jnp.float32),
    )(poe_s[0], Wq, base_rep)
    for s in range(1, SLICES):
        out = pl.pallas_call(
            _mm_body_alias,
            grid=(NW,),
            in_specs=[pl.BlockSpec(memory_space=pltpu.MemorySpace.HBM)]
            + mm_specs,
            out_specs=pl.BlockSpec((RT, OUT_DIM),
                                   lambda t, s=s: (s * NW + t, 0)),
            out_shape=jax.ShapeDtypeStruct((ROWS, OUT_DIM), jnp.float32),
            input_output_aliases={0: 0},
        )(out, poe_s[s], Wq, base_rep)
    return out
